# baseline (speedup 1.0000x reference)
import functools

import jax
import jax.numpy as jnp
from jax import lax
from jax.experimental import pallas as pl
from jax.experimental.pallas import tpu as pltpu
from jax.experimental.pallas import tpu_sc as plsc

B = 4096
NCAT = 26
NCONT = 13
VOCAB = 100000
ED = 32
L1 = 512
L2 = 256
NC = 2
N_EMB = NCAT * ED
EPS = 1e-5

SC_CORES = 2
SC_SUBCORES = 16
NW = SC_CORES * SC_SUBCORES
TOTAL_ROWS = B * NCAT
ROWS_PER_W = TOTAL_ROWS // NW
CHUNK = 128
NCHUNK = ROWS_PER_W // CHUNK
NTILE = NCAT * VOCAB // 4
SLAB = 4 * ED

NBLK = 8
BBLK = B // NBLK
RBLK = BBLK * NCAT
SBLK = RBLK // 128


def _gather_body(table_hbm, slab_hbm, out_hbm, slab_v, slab_buf, sem):
    wid = lax.axis_index("s") * SC_CORES + lax.axis_index("c")
    base = wid * ROWS_PER_W
    pltpu.sync_copy(slab_hbm.at[wid], slab_v)

    def chunk_body(c, carry):
        pltpu.async_copy(table_hbm.at[slab_v.at[c]], slab_buf, sem).wait()
        dst = out_hbm.at[pl.ds(pl.multiple_of(base + c * CHUNK, CHUNK), CHUNK)]
        pltpu.sync_copy(slab_buf, dst)
        return carry

    lax.fori_loop(0, NCHUNK, chunk_body, 0)


@functools.cache
def _sc_gather_fn():
    return pl.kernel(
        _gather_body,
        out_type=jax.ShapeDtypeStruct((TOTAL_ROWS, SLAB), jnp.float32),
        mesh=plsc.VectorSubcoreMesh(
            core_axis_name="c", subcore_axis_name="s",
            num_cores=SC_CORES, num_subcores=SC_SUBCORES,
        ),
        scratch_types=[
            pltpu.VMEM((NCHUNK, CHUNK), jnp.int32),
            pltpu.VMEM((CHUNK, SLAB), jnp.float32),
            pltpu.SemaphoreType.DMA,
        ],
        compiler_params=pltpu.CompilerParams(use_tc_tiling_on_sc=True),
    )


def _select_body(slabs_ref, subp_ref, out_ref):
    f32 = jnp.float32
    v = slabs_ref[...]
    subm = subp_ref[...]
    p = (lax.broadcasted_iota(jnp.int32, (RBLK, SBLK), 0) // 128
         == lax.broadcasted_iota(jnp.int32, (RBLK, SBLK), 1)).astype(f32)
    t = jnp.dot(p, subm, preferred_element_type=f32)
    lane_pick = (lax.broadcasted_iota(jnp.int32, (RBLK, 128), 0) % 128
                 == lax.broadcasted_iota(jnp.int32, (RBLK, 128), 1))
    subcol = jnp.sum(jnp.where(lane_pick, t, 0.0), axis=1, keepdims=True)
    s_of_l = (lax.broadcasted_iota(jnp.int32, (RBLK, SLAB), 1) // ED
              ).astype(f32)
    masked = jnp.where(s_of_l == subcol, v, 0.0)
    sel = jnp.zeros((RBLK, ED), f32)
    for s in range(SLAB // ED):
        sel = sel + masked[:, s * ED:(s + 1) * ED]
    out_ref[...] = sel


def _select(slabs, subp, interpret=False):
    return pl.pallas_call(
        _select_body,
        grid=(NBLK,),
        in_specs=[
            pl.BlockSpec((RBLK, SLAB), lambda i: (i, 0)),
            pl.BlockSpec((SBLK, 128), lambda i: (i, 0)),
        ],
        out_specs=pl.BlockSpec((RBLK, ED), lambda i: (i, 0)),
        out_shape=jax.ShapeDtypeStruct((TOTAL_ROWS, ED), jnp.float32),
        interpret=interpret,
    )(slabs, subp)


def _mlp_body(x1_ref, xc_ref, w1a_ref, w1b_ref, b1_ref, w2_ref, b2_ref,
              w3_ref, b3_ref, g1_ref, be1_ref, g2_ref, be2_ref,
              g3_ref, be3_ref, out_ref):
    f32 = jnp.float32

    def bn(v, g, b):
        m = jnp.mean(v, axis=0, keepdims=True)
        var = jnp.mean((v - m) ** 2, axis=0, keepdims=True)
        return (v - m) * lax.rsqrt(var + EPS) * g + b

    xcn = bn(xc_ref[...], g1_ref[...], be1_ref[...])
    h = (jnp.dot(x1_ref[...], w1a_ref[...], preferred_element_type=f32)
         + jnp.dot(xcn, w1b_ref[...], preferred_element_type=f32)
         + b1_ref[...])
    h = jnp.maximum(h, 0.0)
    h = bn(h, g2_ref[...], be2_ref[...])
    h = jnp.dot(h, w2_ref[...], preferred_element_type=f32) + b2_ref[...]
    h = jnp.maximum(h, 0.0)
    h = bn(h, g3_ref[...], be3_ref[...])
    out_ref[...] = (jnp.dot(h, w3_ref[...], preferred_element_type=f32)
                    + b3_ref[...])


def _mlp(x1, xc, W1a, W1b, b1, W2, b2, W3, b3, g1, be1, g2, be2, g3, be3,
         interpret=False):
    return pl.pallas_call(
        _mlp_body,
        out_shape=jax.ShapeDtypeStruct((B, NC), jnp.float32),
        interpret=interpret,
    )(x1, xc, W1a, W1b, b1.reshape(1, L1), W2, b2.reshape(1, L2),
      W3, b3.reshape(1, NC), g1.reshape(1, NCONT), be1.reshape(1, NCONT),
      g2.reshape(1, L1), be2.reshape(1, L1), g3.reshape(1, L2),
      be3.reshape(1, L2))


def kernel(x, emb_tables, W1, b1, W2, b2, W3, b3, g1, be1, g2, be2, g3, be3):
    table2 = emb_tables.reshape(NTILE, SLAB)
    offsets = (jnp.arange(NCAT, dtype=jnp.int32) * VOCAB)[None, :]
    idx = x[:, :NCAT].astype(jnp.int32) + offsets
    slab3 = (idx >> 2).reshape(NW, NCHUNK, CHUNK)
    subp = (idx & 3).reshape(TOTAL_ROWS // 128, 128).astype(jnp.float32)
    slabs = _sc_gather_fn()(table2, slab3)
    gathered = _select(slabs, subp)
    x1 = gathered.reshape(B, N_EMB)
    xc = x[:, NCAT:]
    return _mlp(x1, xc, W1[:N_EMB], W1[N_EMB:], b1, W2, b2, W3, b3,
                g1, be1, g2, be2, g3, be3)
